# baseline (speedup 1.0000x reference)
import functools

import jax
import jax.numpy as jnp
from jax import lax
from jax.experimental import pallas as pl
from jax.experimental.pallas import tpu as pltpu
from jax.experimental.pallas import tpu_sc as plsc


def _make_gather(D, B, T):
    info = plsc.get_sparse_core_info()
    NC, NS, L = 1, 4, info.num_lanes
    NW = NC * NS
    assert B % NW == 0 and (B // NW) % L == 0
    b_per_w = B // NW
    mesh = plsc.VectorSubcoreMesh(
        core_axis_name="c", subcore_axis_name="s", num_cores=NC, num_subcores=NS
    )

    @functools.partial(
        pl.kernel,
        mesh=mesh,
        out_type=jax.ShapeDtypeStruct((B, D), jnp.float32),
        scratch_types=[
            pltpu.VMEM((b_per_w,), jnp.int32),
            pltpu.VMEM((b_per_w, D), jnp.float32),
            pltpu.SemaphoreType.DMA,
        ],
    )
    def gather(table_hbm, ts_hbm, out_hbm, idx_v, rows_v, sem):
        wid = lax.axis_index("s") * NC + lax.axis_index("c")
        base = wid * b_per_w
        pltpu.sync_copy(ts_hbm.at[pl.ds(base, b_per_w)], idx_v)
        for g in range(b_per_w // L):
            ts = idx_v[pl.ds(g * L, L)]
            rows = base + g * L + lax.iota(jnp.int32, L)
            idx_v[pl.ds(g * L, L)] = rows * T + ts
        pltpu.async_copy(table_hbm.at[idx_v], rows_v, sem).wait()
        pltpu.sync_copy(rows_v, out_hbm.at[pl.ds(base, b_per_w)])

    return gather


def _mm_partial_kernel(el_ref, er_ref, w_ref, b_ref, part_ref):
    D = el_ref.shape[1]
    part_ref[...] = (
        jnp.dot(el_ref[...], w_ref[0:D, :], preferred_element_type=jnp.float32)
        + jnp.dot(er_ref[...], w_ref[D : 2 * D, :], preferred_element_type=jnp.float32)
        + b_ref[...]
    )


def _mm_final_kernel(part_ref, et_ref, w_ref, h_ref):
    D = et_ref.shape[1]
    h_ref[...] = part_ref[...] + jnp.dot(
        et_ref[...], w_ref[2 * D : 3 * D, :], preferred_element_type=jnp.float32
    )


def kernel(inf_enc_seq, inf_enc_key_seq, e_l, e_r, start_ind, end_ind, timestep, W, b):
    B, T, D = inf_enc_seq.shape
    NZ = W.shape[1] // 2
    table = inf_enc_seq.reshape(B * T, D)
    ts = timestep.reshape(B).astype(jnp.int32)
    e_tilde = _make_gather(D, B, T)(table, ts)
    part = pl.pallas_call(
        _mm_partial_kernel,
        out_shape=jax.ShapeDtypeStruct((B, 2 * NZ), jnp.float32),
    )(e_l, e_r, W, b.reshape(1, 2 * NZ))
    h = pl.pallas_call(
        _mm_final_kernel,
        out_shape=jax.ShapeDtypeStruct((B, 2 * NZ), jnp.float32),
    )(part, e_tilde, W)
    return (h[:, :NZ], h[:, NZ:])


# speedup vs baseline: 1.0657x; 1.0657x over previous
import functools

import jax
import jax.numpy as jnp
from jax import lax
from jax.experimental import pallas as pl
from jax.experimental.pallas import tpu as pltpu
from jax.experimental.pallas import tpu_sc as plsc


def _make_gather(D, B, T):
    info = plsc.get_sparse_core_info()
    NC, NS, L = 1, info.num_subcores, info.num_lanes
    NW = NC * NS
    assert B % NW == 0 and (B // NW) % L == 0
    b_per_w = B // NW
    mesh = plsc.VectorSubcoreMesh(
        core_axis_name="c", subcore_axis_name="s", num_cores=NC, num_subcores=NS
    )

    @functools.partial(
        pl.kernel,
        mesh=mesh,
        out_type=jax.ShapeDtypeStruct((B, D), jnp.float32),
        scratch_types=[
            pltpu.VMEM((b_per_w,), jnp.int32),
            pltpu.VMEM((b_per_w, D), jnp.float32),
            pltpu.SemaphoreType.DMA,
        ],
    )
    def gather(table_hbm, ts_hbm, out_hbm, idx_v, rows_v, sem):
        wid = lax.axis_index("s") * NC + lax.axis_index("c")
        base = wid * b_per_w
        pltpu.sync_copy(ts_hbm.at[pl.ds(base, b_per_w)], idx_v)
        for g in range(b_per_w // L):
            ts = idx_v[pl.ds(g * L, L)]
            rows = base + g * L + lax.iota(jnp.int32, L)
            idx_v[pl.ds(g * L, L)] = rows * T + ts
        pltpu.async_copy(table_hbm.at[idx_v], rows_v, sem).wait()
        pltpu.sync_copy(rows_v, out_hbm.at[pl.ds(base, b_per_w)])

    return gather


def _mm_full_kernel(el_ref, er_ref, et_ref, w_ref, b_ref, h_ref):
    D = el_ref.shape[1]
    h_ref[...] = (
        jnp.dot(el_ref[...], w_ref[0:D, :], preferred_element_type=jnp.float32)
        + jnp.dot(er_ref[...], w_ref[D : 2 * D, :], preferred_element_type=jnp.float32)
        + jnp.dot(et_ref[...], w_ref[2 * D : 3 * D, :], preferred_element_type=jnp.float32)
        + b_ref[...]
    )


def kernel(inf_enc_seq, inf_enc_key_seq, e_l, e_r, start_ind, end_ind, timestep, W, b):
    B, T, D = inf_enc_seq.shape
    NZ = W.shape[1] // 2
    table = inf_enc_seq.reshape(B * T, D)
    ts = timestep.reshape(B).astype(jnp.int32)
    e_tilde = _make_gather(D, B, T)(table, ts)
    h = pl.pallas_call(
        _mm_full_kernel,
        out_shape=jax.ShapeDtypeStruct((B, 2 * NZ), jnp.float32),
    )(e_l, e_r, e_tilde, W, b.reshape(1, 2 * NZ))
    return (h[:, :NZ], h[:, NZ:])


# speedup vs baseline: 1.0770x; 1.0106x over previous
import functools

import jax
import jax.numpy as jnp
from jax import lax
from jax.experimental import pallas as pl
from jax.experimental.pallas import tpu as pltpu
from jax.experimental.pallas import tpu_sc as plsc


def _make_gather(D, B, T):
    info = plsc.get_sparse_core_info()
    NC, NS, L = 1, info.num_subcores, info.num_lanes
    NW = NC * NS
    assert B % NW == 0 and (B // NW) % L == 0
    b_per_w = B // NW
    mesh = plsc.VectorSubcoreMesh(
        core_axis_name="c", subcore_axis_name="s", num_cores=NC, num_subcores=NS
    )

    @functools.partial(
        pl.kernel,
        mesh=mesh,
        out_type=jax.ShapeDtypeStruct((B, D), jnp.float32),
        scratch_types=[
            pltpu.VMEM((b_per_w,), jnp.int32),
            pltpu.VMEM((b_per_w, D), jnp.float32),
            pltpu.SemaphoreType.DMA,
            pltpu.SemaphoreType.DMA,
            pltpu.SemaphoreType.DMA,
        ],
    )
    def gather(table_hbm, ts_hbm, out_hbm, idx_v, rows_v, sem_a, sem_b, sem_c):
        wid = lax.axis_index("s") * NC + lax.axis_index("c")
        base = wid * b_per_w
        H = b_per_w // 2
        cp0 = pltpu.make_async_copy(
            ts_hbm.at[pl.ds(base, H)], idx_v.at[pl.ds(0, H)], sem_a
        )
        cp1 = pltpu.make_async_copy(
            ts_hbm.at[pl.ds(base + H, H)], idx_v.at[pl.ds(H, H)], sem_b
        )
        cp0.start()
        cp1.start()
        cp0.wait()
        for g in range(H // L):
            ts = idx_v[pl.ds(g * L, L)]
            rows = base + g * L + lax.iota(jnp.int32, L)
            idx_v[pl.ds(g * L, L)] = rows * T + ts
        g0 = pltpu.make_async_copy(
            table_hbm.at[idx_v.at[pl.ds(0, H)]], rows_v.at[pl.ds(0, H)], sem_c
        )
        g0.start()
        cp1.wait()
        for g in range(H // L, b_per_w // L):
            ts = idx_v[pl.ds(g * L, L)]
            rows = base + g * L + lax.iota(jnp.int32, L)
            idx_v[pl.ds(g * L, L)] = rows * T + ts
        g1 = pltpu.make_async_copy(
            table_hbm.at[idx_v.at[pl.ds(H, H)]], rows_v.at[pl.ds(H, H)], sem_a
        )
        g1.start()
        g0.wait()
        o0 = pltpu.make_async_copy(
            rows_v.at[pl.ds(0, H)], out_hbm.at[pl.ds(base, H)], sem_b
        )
        o0.start()
        g1.wait()
        o1 = pltpu.make_async_copy(
            rows_v.at[pl.ds(H, H)], out_hbm.at[pl.ds(base + H, H)], sem_c
        )
        o1.start()
        o0.wait()
        o1.wait()

    return gather


def _mm_partial_kernel(el_ref, er_ref, w_ref, b_ref, part_ref):
    D = el_ref.shape[1]
    part_ref[...] = (
        jnp.dot(el_ref[...], w_ref[0:D, :], preferred_element_type=jnp.float32)
        + jnp.dot(er_ref[...], w_ref[D : 2 * D, :], preferred_element_type=jnp.float32)
        + b_ref[...]
    )


def _mm_final_kernel(part_ref, et_ref, w_ref, h_ref):
    D = et_ref.shape[1]
    h_ref[...] = part_ref[...] + jnp.dot(
        et_ref[...], w_ref[2 * D : 3 * D, :], preferred_element_type=jnp.float32
    )


def kernel(inf_enc_seq, inf_enc_key_seq, e_l, e_r, start_ind, end_ind, timestep, W, b):
    B, T, D = inf_enc_seq.shape
    NZ = W.shape[1] // 2
    table = inf_enc_seq.reshape(B * T, D)
    ts = timestep.reshape(B).astype(jnp.int32)
    e_tilde = _make_gather(D, B, T)(table, ts)
    part = pl.pallas_call(
        _mm_partial_kernel,
        out_shape=jax.ShapeDtypeStruct((B, 2 * NZ), jnp.float32),
    )(e_l, e_r, W, b.reshape(1, 2 * NZ))
    h = pl.pallas_call(
        _mm_final_kernel,
        out_shape=jax.ShapeDtypeStruct((B, 2 * NZ), jnp.float32),
    )(part, e_tilde, W)
    return (h[:, :NZ], h[:, NZ:])
